# Initial kernel scaffold; baseline (speedup 1.0000x reference)
import jax
import jax.numpy as jnp
from jax.experimental import pallas as pl


def kernel(x, adj, W1, b1, W2, b2):
    raise NotImplementedError("write your pallas kernel here")


# speedup vs baseline: 1.0128x; 1.0128x over previous
import functools

import jax
import jax.numpy as jnp
from jax.experimental import pallas as pl

N = 10000
F = 512
BM = 400


def _mm_kernel(x_ref, w_ref, o_ref):
    o_ref[...] = jnp.dot(x_ref[...], w_ref[...],
                         preferred_element_type=jnp.float32)


def _layer1_kernel(adj_ref, s_ref, b_ref, w2_ref, o_ref):
    acc = jnp.dot(adj_ref[...], s_ref[...],
                  preferred_element_type=jnp.float32)
    h = jnp.maximum(acc + b_ref[...], 0.0)
    o_ref[...] = jnp.dot(h, w2_ref[...], preferred_element_type=jnp.float32)


def _layer2_kernel(adj_ref, s_ref, b_ref, o_ref):
    acc = jnp.dot(adj_ref[...], s_ref[...],
                  preferred_element_type=jnp.float32)
    o_ref[...] = jnp.maximum(acc + b_ref[...], 0.0)


@jax.jit
def kernel(x, adj, W1, b1, W2, b2):
    nblk = N // BM
    b1r = b1.reshape(1, F)
    b2r = b2.reshape(1, F)

    s1 = pl.pallas_call(
        _mm_kernel,
        grid=(nblk,),
        in_specs=[
            pl.BlockSpec((BM, F), lambda i: (i, 0)),
            pl.BlockSpec((F, F), lambda i: (0, 0)),
        ],
        out_specs=pl.BlockSpec((BM, F), lambda i: (i, 0)),
        out_shape=jax.ShapeDtypeStruct((N, F), jnp.float32),
    )(x, W1)

    s2 = pl.pallas_call(
        _layer1_kernel,
        grid=(nblk,),
        in_specs=[
            pl.BlockSpec((BM, N), lambda i: (i, 0)),
            pl.BlockSpec((N, F), lambda i: (0, 0)),
            pl.BlockSpec((1, F), lambda i: (0, 0)),
            pl.BlockSpec((F, F), lambda i: (0, 0)),
        ],
        out_specs=pl.BlockSpec((BM, F), lambda i: (i, 0)),
        out_shape=jax.ShapeDtypeStruct((N, F), jnp.float32),
    )(adj, s1, b1r, W2)

    out = pl.pallas_call(
        _layer2_kernel,
        grid=(nblk,),
        in_specs=[
            pl.BlockSpec((BM, N), lambda i: (i, 0)),
            pl.BlockSpec((N, F), lambda i: (0, 0)),
            pl.BlockSpec((1, F), lambda i: (0, 0)),
        ],
        out_specs=pl.BlockSpec((BM, F), lambda i: (i, 0)),
        out_shape=jax.ShapeDtypeStruct((N, F), jnp.float32),
    )(adj, s2, b2r)

    return out


# speedup vs baseline: 1.0524x; 1.0391x over previous
import functools

import jax
import jax.numpy as jnp
from jax.experimental import pallas as pl

N = 10000
F = 512
BM = 400


def _mm_kernel(x_ref, w_ref, o_ref):
    o_ref[...] = jnp.dot(x_ref[...].astype(jnp.bfloat16),
                         w_ref[...].astype(jnp.bfloat16),
                         preferred_element_type=jnp.float32
                         ).astype(jnp.bfloat16)


def _layer1_kernel(adj_ref, s_ref, b_ref, w2_ref, o_ref):
    acc = jnp.dot(adj_ref[...].astype(jnp.bfloat16), s_ref[...],
                  preferred_element_type=jnp.float32)
    h = jnp.maximum(acc + b_ref[...], 0.0).astype(jnp.bfloat16)
    o_ref[...] = jnp.dot(h, w2_ref[...].astype(jnp.bfloat16),
                         preferred_element_type=jnp.float32
                         ).astype(jnp.bfloat16)


def _layer2_kernel(adj_ref, s_ref, b_ref, o_ref):
    acc = jnp.dot(adj_ref[...].astype(jnp.bfloat16), s_ref[...],
                  preferred_element_type=jnp.float32)
    o_ref[...] = jnp.maximum(acc + b_ref[...], 0.0)


@jax.jit
def kernel(x, adj, W1, b1, W2, b2):
    nblk = N // BM
    b1r = b1.reshape(1, F)
    b2r = b2.reshape(1, F)

    s1 = pl.pallas_call(
        _mm_kernel,
        grid=(nblk,),
        in_specs=[
            pl.BlockSpec((BM, F), lambda i: (i, 0)),
            pl.BlockSpec((F, F), lambda i: (0, 0)),
        ],
        out_specs=pl.BlockSpec((BM, F), lambda i: (i, 0)),
        out_shape=jax.ShapeDtypeStruct((N, F), jnp.bfloat16),
    )(x, W1)

    s2 = pl.pallas_call(
        _layer1_kernel,
        grid=(nblk,),
        in_specs=[
            pl.BlockSpec((BM, N), lambda i: (i, 0)),
            pl.BlockSpec((N, F), lambda i: (0, 0)),
            pl.BlockSpec((1, F), lambda i: (0, 0)),
            pl.BlockSpec((F, F), lambda i: (0, 0)),
        ],
        out_specs=pl.BlockSpec((BM, F), lambda i: (i, 0)),
        out_shape=jax.ShapeDtypeStruct((N, F), jnp.bfloat16),
    )(adj, s1, b1r, W2)

    out = pl.pallas_call(
        _layer2_kernel,
        grid=(nblk,),
        in_specs=[
            pl.BlockSpec((BM, N), lambda i: (i, 0)),
            pl.BlockSpec((N, F), lambda i: (0, 0)),
            pl.BlockSpec((1, F), lambda i: (0, 0)),
        ],
        out_specs=pl.BlockSpec((BM, F), lambda i: (i, 0)),
        out_shape=jax.ShapeDtypeStruct((N, F), jnp.float32),
    )(adj, s2, b2r)

    return out
